# baseline (device time: 242652 ns/iter reference)
import jax
import jax.numpy as jnp
from jax import lax
from jax.experimental import pallas as pl
from jax.experimental.pallas import tpu as pltpu

N_DEV = 4


def kernel(x, w_mat):
    m, k_shard = x.shape
    _, n = w_mat.shape
    mc = m // N_DEV
    nh = n // 2

    def body(x_ref, w_ref, out_ref, x_stage, recv_a, recv_b, q_ref, amax_ref,
             copy_sem, rsa_send, rsa_recv, rsb_send, rsb_recv,
             aga_send, aga_recv, agb_send, agb_recv, amx_send, amx_recv,
             credit_a, credit_b):
        me = lax.axis_index("i")
        left = (me + N_DEV - 1) % N_DEV
        right = (me + 1) % N_DEV
        MESH = pl.DeviceIdType.MESH

        barrier_sem = pltpu.get_barrier_semaphore()
        for nbr in (left, right):
            pl.semaphore_signal(barrier_sem, inc=1, device_id=(nbr,),
                                device_id_type=MESH)
        pl.semaphore_wait(barrier_sem, 2)

        def gemm_half(c, col0):
            for h in range(2):
                r0 = c * mc + h * (mc // 2)
                cp = pltpu.make_async_copy(
                    x_ref.at[pl.ds(r0, mc // 2), :], x_stage, copy_sem)
                cp.start()
                cp.wait()
                out_ref[pl.ds(r0, mc // 2), pl.ds(col0, nh)] = jnp.dot(
                    x_stage[:, :], w_ref[:, pl.ds(col0, nh)],
                    preferred_element_type=jnp.float32)

        def make_rs_a(s):
            return pltpu.make_async_remote_copy(
                src_ref=out_ref.at[
                    pl.ds(((me - s) % N_DEV) * mc, mc), pl.ds(0, nh)],
                dst_ref=recv_a,
                send_sem=rsa_send.at[s], recv_sem=rsa_recv.at[s],
                device_id=(right,), device_id_type=MESH)

        def make_rs_b(s):
            return pltpu.make_async_remote_copy(
                src_ref=out_ref.at[
                    pl.ds(((me + s) % N_DEV) * mc, mc), pl.ds(nh, nh)],
                dst_ref=recv_b,
                send_sem=rsb_send.at[s], recv_sem=rsb_recv.at[s],
                device_id=(left,), device_id_type=MESH)

        def acc_a(s):
            c = (me - s - 1) % N_DEV
            v = out_ref[pl.ds(c * mc, mc), pl.ds(0, nh)] + recv_a[:, :]
            out_ref[pl.ds(c * mc, mc), pl.ds(0, nh)] = v
            return v

        def acc_b(s):
            c = (me + s + 1) % N_DEV
            v = out_ref[pl.ds(c * mc, mc), pl.ds(nh, nh)] + recv_b[:, :]
            out_ref[pl.ds(c * mc, mc), pl.ds(nh, nh)] = v
            return v

        far = (me + 2) % N_DEV

        gemm_half(me, 0)
        a0 = make_rs_a(0)
        a0.start()
        gemm_half(me, nh)
        b0 = make_rs_b(0)
        b0.start()
        gemm_half(left, 0)
        gemm_half(right, nh)

        a0.wait()
        acc_a(0)
        pl.semaphore_signal(credit_a, inc=1, device_id=(left,),
                            device_id_type=MESH)
        pl.semaphore_wait(credit_a, 1)
        a1 = make_rs_a(1)
        a1.start()
        gemm_half(far, 0)

        b0.wait()
        acc_b(0)
        pl.semaphore_signal(credit_b, inc=1, device_id=(right,),
                            device_id_type=MESH)
        pl.semaphore_wait(credit_b, 1)
        b1 = make_rs_b(1)
        b1.start()
        gemm_half(far, nh)

        a1.wait()
        acc_a(1)
        pl.semaphore_signal(credit_a, inc=1, device_id=(left,),
                            device_id_type=MESH)
        pl.semaphore_wait(credit_a, 1)
        a2 = make_rs_a(2)
        a2.start()
        gemm_half(right, 0)

        b1.wait()
        acc_b(1)
        pl.semaphore_signal(credit_b, inc=1, device_id=(right,),
                            device_id_type=MESH)
        pl.semaphore_wait(credit_b, 1)
        b2 = make_rs_b(2)
        b2.start()
        gemm_half(left, nh)

        a2.wait()
        amax_a = jnp.max(acc_a(2))
        b2.wait()
        amax_b = jnp.max(acc_b(2))

        own_a = right
        own_b = left

        amax_loc = jnp.maximum(jnp.maximum(amax_a, amax_b), 0.0)
        amax_ref[pl.ds(me, 1), :, :] = jnp.full(
            (1, 8, 128), amax_loc, jnp.float32)
        peers = [(me + 1) % N_DEV, (me + 2) % N_DEV, (me + 3) % N_DEV]
        sends = []
        for p in peers:
            d = pltpu.make_async_remote_copy(
                src_ref=amax_ref.at[me], dst_ref=amax_ref.at[me],
                send_sem=amx_send.at[p], recv_sem=amx_recv.at[me],
                device_id=(p,), device_id_type=MESH)
            d.start()
            sends.append(d)
        for p in peers:
            pltpu.make_async_remote_copy(
                src_ref=amax_ref.at[p], dst_ref=amax_ref.at[p],
                send_sem=amx_send.at[p], recv_sem=amx_recv.at[p],
                device_id=(p,), device_id_type=MESH).wait_recv()
        for d in sends:
            d.wait_send()
        scale = jnp.max(amax_ref[:, :, :]) / 127.0

        def quant(c, col0):
            y = jnp.maximum(out_ref[pl.ds(c * mc, mc), pl.ds(col0, nh)], 0.0)
            q = jnp.clip(jnp.round(y / scale), -127.0, 127.0)
            q_ref[pl.ds(c * mc, mc), pl.ds(col0, nh)] = q.astype(jnp.int8)

        def dequant(c, col0):
            out_ref[pl.ds(c * mc, mc), pl.ds(col0, nh)] = (
                q_ref[pl.ds(c * mc, mc), pl.ds(col0, nh)].astype(jnp.float32)
                * scale)

        def make_ag_a(s):
            c = (me + 1 - s) % N_DEV
            return pltpu.make_async_remote_copy(
                src_ref=q_ref.at[pl.ds(c * mc, mc), pl.ds(0, nh)],
                dst_ref=q_ref.at[pl.ds(c * mc, mc), pl.ds(0, nh)],
                send_sem=aga_send.at[s], recv_sem=aga_recv.at[s],
                device_id=(right,), device_id_type=MESH)

        def make_ag_b(s):
            c = (me - 1 + s) % N_DEV
            return pltpu.make_async_remote_copy(
                src_ref=q_ref.at[pl.ds(c * mc, mc), pl.ds(nh, nh)],
                dst_ref=q_ref.at[pl.ds(c * mc, mc), pl.ds(nh, nh)],
                send_sem=agb_send.at[s], recv_sem=agb_recv.at[s],
                device_id=(left,), device_id_type=MESH)

        quant(own_a, 0)
        ag_a0 = make_ag_a(0)
        ag_a0.start()
        quant(own_b, nh)
        ag_b0 = make_ag_b(0)
        ag_b0.start()
        dequant(own_a, 0)
        dequant(own_b, nh)
        ag_a0.wait()
        ag_b0.wait()

        ag_a1 = make_ag_a(1)
        ag_a1.start()
        ag_b1 = make_ag_b(1)
        ag_b1.start()
        dequant(me, 0)
        dequant(me, nh)
        ag_a1.wait()
        ag_b1.wait()

        ag_a2 = make_ag_a(2)
        ag_a2.start()
        ag_b2 = make_ag_b(2)
        ag_b2.start()
        dequant(left, 0)
        dequant(right, nh)
        ag_a2.wait()
        ag_b2.wait()
        dequant(far, 0)
        dequant(far, nh)

    return pl.pallas_call(
        body,
        out_shape=jax.ShapeDtypeStruct((m, n), jnp.float32),
        in_specs=[
            pl.BlockSpec(memory_space=pl.ANY),
            pl.BlockSpec(memory_space=pltpu.VMEM),
        ],
        out_specs=pl.BlockSpec(memory_space=pltpu.VMEM),
        scratch_shapes=[
            pltpu.VMEM((mc // 2, k_shard), jnp.float32),
            pltpu.VMEM((mc, nh), jnp.float32),
            pltpu.VMEM((mc, nh), jnp.float32),
            pltpu.VMEM((m, n), jnp.int8),
            pltpu.VMEM((N_DEV, 8, 128), jnp.float32),
            pltpu.SemaphoreType.DMA,
            pltpu.SemaphoreType.DMA((N_DEV - 1,)),
            pltpu.SemaphoreType.DMA((N_DEV - 1,)),
            pltpu.SemaphoreType.DMA((N_DEV - 1,)),
            pltpu.SemaphoreType.DMA((N_DEV - 1,)),
            pltpu.SemaphoreType.DMA((N_DEV - 1,)),
            pltpu.SemaphoreType.DMA((N_DEV - 1,)),
            pltpu.SemaphoreType.DMA((N_DEV - 1,)),
            pltpu.SemaphoreType.DMA((N_DEV - 1,)),
            pltpu.SemaphoreType.DMA((N_DEV,)),
            pltpu.SemaphoreType.DMA((N_DEV,)),
            pltpu.SemaphoreType.REGULAR,
            pltpu.SemaphoreType.REGULAR,
        ],
        compiler_params=pltpu.CompilerParams(
            collective_id=0,
            vmem_limit_bytes=63 * 1024 * 1024,
        ),
    )(x, w_mat)


# device time: 242259 ns/iter; 1.0016x vs baseline; 1.0016x over previous
import jax
import jax.numpy as jnp
from jax import lax
from jax.experimental import pallas as pl
from jax.experimental.pallas import tpu as pltpu

N_DEV = 4


def kernel(x, w_mat):
    m, k_shard = x.shape
    _, n = w_mat.shape
    mc = m // N_DEV
    mh = mc // 2
    nh = n // 2

    def body(x_ref, w_ref, out_ref, x_stage, recv_a, recv_b, q_ref, amax_ref,
             copy_sem, rsa_send, rsa_recv, rsb_send, rsb_recv,
             aga_send, aga_recv, agb_send, agb_recv, amx_send, amx_recv,
             cred_a, cred_b):
        me = lax.axis_index("i")
        left = (me + N_DEV - 1) % N_DEV
        right = (me + 1) % N_DEV
        far = (me + 2) % N_DEV
        MESH = pl.DeviceIdType.MESH

        barrier_sem = pltpu.get_barrier_semaphore()
        for nbr in (left, right):
            pl.semaphore_signal(barrier_sem, inc=1, device_id=(nbr,),
                                device_id_type=MESH)
        pl.semaphore_wait(barrier_sem, 2)

        def gemm_q(c, col0, j):
            r0 = c * mc + j * mh
            cp = pltpu.make_async_copy(
                x_ref.at[pl.ds(r0, mh), :], x_stage, copy_sem)
            cp.start()
            cp.wait()
            out_ref[pl.ds(r0, mh), pl.ds(col0, nh)] = jnp.dot(
                x_stage[:, :], w_ref[:, pl.ds(col0, nh)],
                preferred_element_type=jnp.float32)

        def rs_a(s, j):
            c = (me - s) % N_DEV
            return pltpu.make_async_remote_copy(
                src_ref=out_ref.at[pl.ds(c * mc + j * mh, mh), pl.ds(0, nh)],
                dst_ref=recv_a.at[j],
                send_sem=rsa_send.at[2 * s + j],
                recv_sem=rsa_recv.at[2 * s + j],
                device_id=(right,), device_id_type=MESH)

        def rs_b(s, j):
            c = (me + s) % N_DEV
            return pltpu.make_async_remote_copy(
                src_ref=out_ref.at[pl.ds(c * mc + j * mh, mh), pl.ds(nh, nh)],
                dst_ref=recv_b.at[j],
                send_sem=rsb_send.at[2 * s + j],
                recv_sem=rsb_recv.at[2 * s + j],
                device_id=(left,), device_id_type=MESH)

        def acc_a(s, j):
            c = (me - s - 1) % N_DEV
            r0 = c * mc + j * mh
            v = out_ref[pl.ds(r0, mh), pl.ds(0, nh)] + recv_a[j, :, :]
            out_ref[pl.ds(r0, mh), pl.ds(0, nh)] = v
            return v

        def acc_b(s, j):
            c = (me + s + 1) % N_DEV
            r0 = c * mc + j * mh
            v = out_ref[pl.ds(r0, mh), pl.ds(nh, nh)] + recv_b[j, :, :]
            out_ref[pl.ds(r0, mh), pl.ds(nh, nh)] = v
            return v

        def consume_a(s, j):
            rs_a(s, j).wait()
            v = acc_a(s, j)
            if s < N_DEV - 2:
                pl.semaphore_signal(cred_a.at[j], inc=1, device_id=(left,),
                                    device_id_type=MESH)
                pl.semaphore_wait(cred_a.at[j], 1)
                rs_a(s + 1, j).start()
            return v

        def consume_b(s, j):
            rs_b(s, j).wait()
            v = acc_b(s, j)
            if s < N_DEV - 2:
                pl.semaphore_signal(cred_b.at[j], inc=1, device_id=(right,),
                                    device_id_type=MESH)
                pl.semaphore_wait(cred_b.at[j], 1)
                rs_b(s + 1, j).start()
            return v

        gemm_q(me, 0, 0)
        rs_a(0, 0).start()
        gemm_q(me, 0, 1)
        rs_a(0, 1).start()
        gemm_q(me, nh, 0)
        rs_b(0, 0).start()
        gemm_q(me, nh, 1)
        rs_b(0, 1).start()

        gemm_q(left, 0, 0)
        gemm_q(left, 0, 1)
        gemm_q(right, nh, 0)
        gemm_q(right, nh, 1)

        consume_a(0, 0)
        gemm_q(far, 0, 0)
        consume_a(0, 1)
        gemm_q(far, 0, 1)
        consume_b(0, 0)
        gemm_q(far, nh, 0)
        consume_b(0, 1)
        gemm_q(far, nh, 1)

        consume_a(1, 0)
        gemm_q(right, 0, 0)
        consume_a(1, 1)
        gemm_q(right, 0, 1)
        consume_b(1, 0)
        gemm_q(left, nh, 0)
        consume_b(1, 1)
        gemm_q(left, nh, 1)

        amax_v = jnp.float32(0.0)
        amax_v = jnp.maximum(amax_v, jnp.max(consume_a(2, 0)))
        amax_v = jnp.maximum(amax_v, jnp.max(consume_a(2, 1)))
        amax_v = jnp.maximum(amax_v, jnp.max(consume_b(2, 0)))
        amax_v = jnp.maximum(amax_v, jnp.max(consume_b(2, 1)))

        own_a = right
        own_b = left
        peers = [right, far, (me + 3) % N_DEV]

        amax_ref[pl.ds(me, 1), :, :] = jnp.full(
            (1, 8, 128), amax_v, jnp.float32)
        amx_sends = []
        for p in peers:
            d = pltpu.make_async_remote_copy(
                src_ref=amax_ref.at[me], dst_ref=amax_ref.at[me],
                send_sem=amx_send.at[p], recv_sem=amx_recv.at[me],
                device_id=(p,), device_id_type=MESH)
            d.start()
            amx_sends.append(d)
        for p in peers:
            pltpu.make_async_remote_copy(
                src_ref=amax_ref.at[p], dst_ref=amax_ref.at[p],
                send_sem=amx_send.at[p], recv_sem=amx_recv.at[p],
                device_id=(p,), device_id_type=MESH).wait_recv()
        for d in amx_sends:
            d.wait_send()
        scale = jnp.max(amax_ref[:, :, :]) / 127.0

        def quant(c, col0):
            y = jnp.maximum(out_ref[pl.ds(c * mc, mc), pl.ds(col0, nh)], 0.0)
            q = jnp.clip(jnp.round(y / scale), -127.0, 127.0)
            q_ref[pl.ds(c * mc, mc), pl.ds(col0, nh)] = q.astype(jnp.int8)

        def dequant(c, col0):
            out_ref[pl.ds(c * mc, mc), pl.ds(col0, nh)] = (
                q_ref[pl.ds(c * mc, mc), pl.ds(col0, nh)].astype(jnp.float32)
                * scale)

        def ag_a(src_dev, p):
            c = (src_dev + 1) % N_DEV
            return pltpu.make_async_remote_copy(
                src_ref=q_ref.at[pl.ds(c * mc, mc), pl.ds(0, nh)],
                dst_ref=q_ref.at[pl.ds(c * mc, mc), pl.ds(0, nh)],
                send_sem=aga_send.at[p], recv_sem=aga_recv.at[src_dev],
                device_id=(p,), device_id_type=MESH)

        def ag_b(src_dev, p):
            c = (src_dev + N_DEV - 1) % N_DEV
            return pltpu.make_async_remote_copy(
                src_ref=q_ref.at[pl.ds(c * mc, mc), pl.ds(nh, nh)],
                dst_ref=q_ref.at[pl.ds(c * mc, mc), pl.ds(nh, nh)],
                send_sem=agb_send.at[p], recv_sem=agb_recv.at[src_dev],
                device_id=(p,), device_id_type=MESH)

        quant(own_a, 0)
        quant(own_b, nh)
        ag_sends = []
        for p in peers:
            da = ag_a(me, p)
            da.start()
            db = ag_b(me, p)
            db.start()
            ag_sends.extend((da, db))

        dequant(own_a, 0)
        dequant(own_b, nh)
        for p in [right, left, far]:
            ag_a(p, p).wait_recv()
            dequant((p + 1) % N_DEV, 0)
            ag_b(p, p).wait_recv()
            dequant((p + N_DEV - 1) % N_DEV, nh)
        for d in ag_sends:
            d.wait_send()

    return pl.pallas_call(
        body,
        out_shape=jax.ShapeDtypeStruct((m, n), jnp.float32),
        in_specs=[
            pl.BlockSpec(memory_space=pl.ANY),
            pl.BlockSpec(memory_space=pltpu.VMEM),
        ],
        out_specs=pl.BlockSpec(memory_space=pltpu.VMEM),
        scratch_shapes=[
            pltpu.VMEM((mc // 2, k_shard), jnp.float32),
            pltpu.VMEM((2, mc // 2, nh), jnp.float32),
            pltpu.VMEM((2, mc // 2, nh), jnp.float32),
            pltpu.VMEM((m, n), jnp.int8),
            pltpu.VMEM((N_DEV, 8, 128), jnp.float32),
            pltpu.SemaphoreType.DMA,
            pltpu.SemaphoreType.DMA((2 * (N_DEV - 1),)),
            pltpu.SemaphoreType.DMA((2 * (N_DEV - 1),)),
            pltpu.SemaphoreType.DMA((2 * (N_DEV - 1),)),
            pltpu.SemaphoreType.DMA((2 * (N_DEV - 1),)),
            pltpu.SemaphoreType.DMA((N_DEV,)),
            pltpu.SemaphoreType.DMA((N_DEV,)),
            pltpu.SemaphoreType.DMA((N_DEV,)),
            pltpu.SemaphoreType.DMA((N_DEV,)),
            pltpu.SemaphoreType.DMA((N_DEV,)),
            pltpu.SemaphoreType.DMA((N_DEV,)),
            pltpu.SemaphoreType.REGULAR((2,)),
            pltpu.SemaphoreType.REGULAR((2,)),
        ],
        compiler_params=pltpu.CompilerParams(
            collective_id=0,
            vmem_limit_bytes=63 * 1024 * 1024,
        ),
    )(x, w_mat)
